# baseline (speedup 1.0000x reference)
import jax
import jax.numpy as jnp
from jax.experimental import pallas as pl
from jax.experimental.pallas import tpu as pltpu

D = 4096
K = 4
R1 = 128 // K
R2 = 256 // K
R3 = D // K

_DN = (((1,), (1,)), ((), ()))


def _body(xt_ref, b1_ref, W2_ref, b2_ref, bc1_ref, bc2_ref, reg_ref,
          W1_hbm, Wc1_hbm, Wc2_hbm,
          imp_ref, cons_ref, loss_ref,
          w1_v, wc1_v, wc2_v, sems):
    copies = []
    for k in range(K):
        copies.append(pltpu.make_async_copy(
            W1_hbm.at[pl.ds(k * R1, R1), :], w1_v.at[pl.ds(k * R1, R1), :],
            sems.at[k]))
    for k in range(K):
        copies.append(pltpu.make_async_copy(
            Wc1_hbm.at[pl.ds(k * R2, R2), :], wc1_v.at[pl.ds(k * R2, R2), :],
            sems.at[K + k]))
    for k in range(K):
        copies.append(pltpu.make_async_copy(
            Wc2_hbm.at[pl.ds(k * R3, R3), :], wc2_v.at[pl.ds(k * R3, R3), :],
            sems.at[2 * K + k]))
    for c in copies:
        c.start()

    xt = xt_ref[...]
    x = xt[:, :D]

    for k in range(K):
        copies[k].wait()
    h = jax.lax.dot_general(xt, w1_v[...], _DN,
                            preferred_element_type=jnp.float32)
    h = jnp.maximum(h + b1_ref[...], 0.0)
    logit = jnp.sum(h * W2_ref[...]) + b2_ref[0]
    imp = jax.nn.sigmoid(logit)
    imp_ref[0] = imp
    gate = jnp.where(imp > 0.5, jnp.float32(1.0), jnp.float32(0.0))

    for k in range(K):
        copies[K + k].wait()
    hc = jax.lax.dot_general(x, wc1_v[...], _DN,
                             preferred_element_type=jnp.float32)
    hc = jnp.maximum(hc + bc1_ref[...], 0.0)

    for k in range(K):
        copies[2 * K + k].wait()
        cons_k = jax.lax.dot_general(hc, wc2_v[pl.ds(k * R3, R3), :], _DN,
                                     preferred_element_type=jnp.float32)
        cons_k = cons_k + bc2_ref[:, pl.ds(k * R3, R3)]
        cons_ref[:, pl.ds(k * R3, R3)] = cons_k * gate

    dist = jnp.sum((x - x) ** 2)
    loss_ref[0] = jnp.where(imp > 0.5, reg_ref[0] * (imp * dist),
                            jnp.float32(0.0))


def kernel(current_features, target, W1, b1, W2, b2, Wc1, bc1, Wc2, bc2,
           reg_controller):
    xt = jnp.concatenate([current_features, target]).reshape(1, 2 * D)
    smem = pl.BlockSpec(memory_space=pltpu.SMEM)
    hbm = pl.BlockSpec(memory_space=pl.ANY)
    imp, cons, loss = pl.pallas_call(
        _body,
        out_shape=(
            jax.ShapeDtypeStruct((1,), jnp.float32),
            jax.ShapeDtypeStruct((1, D), jnp.float32),
            jax.ShapeDtypeStruct((1,), jnp.float32),
        ),
        in_specs=[pl.BlockSpec((1, 2 * D), lambda: (0, 0)),
                  pl.BlockSpec((1, 128), lambda: (0, 0)),
                  pl.BlockSpec((1, 128), lambda: (0, 0)),
                  smem,
                  pl.BlockSpec((1, 256), lambda: (0, 0)),
                  pl.BlockSpec((1, D), lambda: (0, 0)),
                  smem,
                  hbm, hbm, hbm],
        out_specs=(smem,
                   pl.BlockSpec((1, D), lambda: (0, 0)),
                   smem),
        scratch_shapes=[pltpu.VMEM((128, 2 * D), jnp.float32),
                        pltpu.VMEM((256, D), jnp.float32),
                        pltpu.VMEM((D, 256), jnp.float32),
                        pltpu.SemaphoreType.DMA((3 * K,))],
    )(xt, b1.reshape(1, 128), W2, b2,
      bc1.reshape(1, 256), bc2.reshape(1, D), reg_controller.reshape(1),
      W1, Wc1, Wc2)
    return imp, cons.reshape(D), loss.reshape(())
